# baseline (speedup 1.0000x reference)
import functools

import jax
import jax.numpy as jnp
from jax import lax
from jax.experimental import pallas as pl
from jax.experimental.pallas import tpu as pltpu
from jax.experimental.pallas import tpu_sc as plsc

B = 16384
F = 26
V = 1000000
D = 16

NC = 2
NS = 16
NW = NC * NS
B_PER_W = B // NW
IDX_COLS = 104
IDX_ROWS = (B_PER_W * F) // IDX_COLS
CB = 32
ROWS_PER_CHUNK = CB * F
GROWS = ROWS_PER_CHUNK // IDX_COLS
NCHUNK = B_PER_W // CB
NBUF = 4

TW = 8192
TGRID = pl.cdiv(V, TW)
TREM = V - (TGRID - 1) * TW


def _tr_body(in_ref, out_hbm, y_ref, sem):
    c = pl.program_id(0)
    y_ref[...] = in_ref[...].T

    @pl.when(c < TGRID - 1)
    def _():
        pltpu.async_copy(y_ref, out_hbm.at[pl.ds(c * TW, TW), :], sem).wait()

    @pl.when(c == TGRID - 1)
    def _():
        pltpu.async_copy(y_ref.at[pl.ds(0, TREM), :],
                         out_hbm.at[pl.ds(c * TW, TREM), :], sem).wait()


_transpose_tc = pl.pallas_call(
    _tr_body,
    grid=(TGRID,),
    in_specs=[pl.BlockSpec((D, TW), lambda c: (0, c))],
    out_specs=pl.BlockSpec(memory_space=pl.ANY),
    out_shape=jax.ShapeDtypeStruct((V, D), jnp.float32),
    scratch_shapes=[pltpu.VMEM((TW, D), jnp.float32), pltpu.SemaphoreType.DMA],
)

_mesh = plsc.VectorSubcoreMesh(core_axis_name="c", subcore_axis_name="s")


@functools.partial(
    pl.kernel,
    out_type=jax.ShapeDtypeStruct((B,), jnp.float32),
    mesh=_mesh,
    compiler_params=pltpu.CompilerParams(needs_layout_passes=False, use_tc_tiling_on_sc=False),
    scratch_types=[
        pltpu.VMEM((IDX_ROWS, IDX_COLS), jnp.int32),
        [pltpu.VMEM((ROWS_PER_CHUNK, D), jnp.float32) for _ in range(NBUF)],
        pltpu.VMEM((B_PER_W * F + 16,), jnp.float32),
        pltpu.VMEM((256,), jnp.float32),
        pltpu.VMEM((B_PER_W,), jnp.float32),
        pltpu.VMEM((16,), jnp.float32),
        [pltpu.SemaphoreType.DMA for _ in range(NBUF)],
        pltpu.SemaphoreType.DMA,
    ],
)
def _fm_sc(idx_hbm, fm_hbm, lr_hbm, bias_hbm, out_hbm,
           idx_v, rows_bufs, lr_v, tm_v, out_v, bias_v, sems, sem_lr):
    wid = lax.axis_index("s") * NC + lax.axis_index("c")
    pltpu.sync_copy(idx_hbm.at[wid], idx_v)
    pltpu.sync_copy(bias_hbm, bias_v.at[pl.ds(0, 1)])
    for r in range(IDX_ROWS):
        pltpu.async_copy(lr_hbm.at[idx_v.at[r]],
                         lr_v.at[pl.ds(r * IDX_COLS, IDX_COLS)], sem_lr)
    bias_s = bias_v[pl.ds(0, 16)][0]
    lane = lax.iota(jnp.int32, 16)
    mask10 = lane < 10
    zero16 = jnp.zeros((16,), jnp.float32)

    def fire(c, buf, sem):
        for j in range(GROWS):
            pltpu.async_copy(fm_hbm.at[idx_v.at[c * GROWS + j]],
                             buf.at[pl.ds(j * IDX_COLS, IDX_COLS)], sem)

    def drain(buf, sem):
        pltpu.make_async_copy(fm_hbm.at[pl.ds(0, ROWS_PER_CHUNK)], buf, sem).wait()

    def compute(c, buf):
        for g in range(CB // 16):
            for bb in range(16):
                b = g * 16 + bb
                s = zero16
                q = zero16
                for f in range(F):
                    e = buf[b * F + f]
                    s = s + e
                    q = q + e * e
                t = 0.5 * (s * s - q)
                l1 = lr_v[pl.ds(c * (CB * F) + b * F, 16)]
                l2 = jnp.where(mask10,
                               lr_v[pl.ds(c * (CB * F) + b * F + 16, 16)], 0.0)
                tm_v[pl.ds(bb * 16, 16)] = t + l1 + l2
            acc = jnp.full((16,), bias_s, jnp.float32)
            for dcol in range(16):
                acc = acc + plsc.load_gather(tm_v, [lane * 16 + dcol])
            out_v[pl.ds(c * CB + g * 16, 16)] = acc

    for p in range(NBUF):
        fire(p, rows_bufs[p], sems[p])
    pltpu.make_async_copy(lr_hbm.at[pl.ds(0, B_PER_W * F)],
                          lr_v.at[pl.ds(0, B_PER_W * F)], sem_lr).wait()

    def body(i, carry):
        c0 = i * NBUF
        for p in range(NBUF):
            c = c0 + p
            drain(rows_bufs[p], sems[p])
            compute(c, rows_bufs[p])

            @pl.when(c + NBUF < NCHUNK)
            def _():
                fire(c + NBUF, rows_bufs[p], sems[p])
        return carry

    lax.fori_loop(0, NCHUNK // NBUF, body, 0)
    pltpu.sync_copy(out_v, out_hbm.at[pl.ds(wid * B_PER_W, B_PER_W)])


def kernel(cate_indices, fm_table, lr_table, lr_bias):
    idx = cate_indices.astype(jnp.int32).reshape(NW, IDX_ROWS, IDX_COLS)
    fm_rows = _transpose_tc(fm_table.T)
    lr_flat = lr_table.reshape(V)
    out = _fm_sc(idx, fm_rows, lr_flat, lr_bias)
    return out.reshape(B, 1)


# speedup vs baseline: 2.9593x; 2.9593x over previous
import functools

import jax
import jax.numpy as jnp
from jax import lax
from jax.experimental import pallas as pl
from jax.experimental.pallas import tpu as pltpu
from jax.experimental.pallas import tpu_sc as plsc

B = 16384
F = 26
V = 1000000
D = 16

NC = 2
NS = 16
NW = NC * NS
B_PER_W = B // NW
IDX_COLS = 104
IDX_ROWS = (B_PER_W * F) // IDX_COLS
CB = 32
ROWS_PER_CHUNK = CB * F
GROWS = ROWS_PER_CHUNK // IDX_COLS
NCHUNK = B_PER_W // CB
NBUF = 4

_mesh = plsc.VectorSubcoreMesh(core_axis_name="c", subcore_axis_name="s")

TTILES = (V + 127) // 128
TPW = (TTILES + NW - 1) // NW


@functools.partial(
    pl.kernel,
    out_type=jax.ShapeDtypeStruct((V * D,), jnp.float32),
    mesh=_mesh,
    compiler_params=pltpu.CompilerParams(needs_layout_passes=False, use_tc_tiling_on_sc=True),
    scratch_types=[
        [pltpu.VMEM((8, 128), jnp.float32) for _ in range(4)],
        [pltpu.VMEM((2048,), jnp.float32) for _ in range(2)],
        [pltpu.SemaphoreType.DMA for _ in range(2)],
        [pltpu.SemaphoreType.DMA for _ in range(2)],
    ],
)
def _tr_sc(fmt_hbm, out_hbm, xbufs, ybufs, insems, outsems):
    wid = lax.axis_index("s") * NC + lax.axis_index("c")
    lane16 = lax.iota(jnp.int32, 16) * 16

    def fire_in(tc, p):
        pltpu.async_copy(fmt_hbm.at[pl.ds(0, 8), pl.ds(tc * 128, 128)],
                         xbufs[2 * p], insems[p])
        pltpu.async_copy(fmt_hbm.at[pl.ds(8, 8), pl.ds(tc * 128, 128)],
                         xbufs[2 * p + 1], insems[p])

    def shuffle(p):
        y = ybufs[p]
        for d in range(D):
            src = xbufs[2 * p + (d // 8)]
            for v in range(8):
                va = src[d % 8, pl.ds(v * 16, 16)]
                plsc.store_scatter(y, [lane16 + (v * 256 + d)], va)

    def drain_in(p):
        pltpu.make_async_copy(fmt_hbm.at[pl.ds(0, 8), pl.ds(0, 128)],
                              xbufs[2 * p], insems[p]).wait()
        pltpu.make_async_copy(fmt_hbm.at[pl.ds(0, 8), pl.ds(0, 128)],
                              xbufs[2 * p + 1], insems[p]).wait()

    def write_out(tc, p):
        @pl.when(tc < TTILES - 1)
        def _():
            pltpu.async_copy(ybufs[p], out_hbm.at[pl.ds(tc * 2048, 2048)],
                             outsems[p])

        @pl.when(tc == TTILES - 1)
        def _():
            pltpu.async_copy(ybufs[p].at[pl.ds(0, 1024)],
                             out_hbm.at[pl.ds(tc * 2048, 1024)], outsems[p])

    def drain_out(tc, p):
        @pl.when(tc < TTILES - 1)
        def _():
            pltpu.make_async_copy(ybufs[p], out_hbm.at[pl.ds(0, 2048)],
                                  outsems[p]).wait()

        @pl.when(tc == TTILES - 1)
        def _():
            pltpu.make_async_copy(ybufs[p].at[pl.ds(0, 1024)],
                                  out_hbm.at[pl.ds(0, 1024)], outsems[p]).wait()

    @pl.when(wid < TTILES)
    def _():
        fire_in(wid, 0)

    def tr_body(k, carry):
        for p in range(2):
            tc = (2 * k + p) * NW + wid
            nxt = tc + NW

            @pl.when(tc < TTILES)
            def _():
                drain_in(p)

                @pl.when(nxt < TTILES)
                def _():
                    fire_in(nxt, 1 - p)

                @pl.when(tc >= 2 * NW)
                def _():
                    drain_out(tc - 2 * NW, p)

                shuffle(p)
                write_out(tc, p)
        return carry

    lax.fori_loop(0, (TPW + 1) // 2, tr_body, 0)
    n_steps = (TTILES - 1 - wid) // NW + 1
    for p in range(2):
        m_p = n_steps - 1 - jnp.mod(n_steps - 1 - p, 2)

        @pl.when(m_p >= 0)
        def _():
            drain_out(m_p * NW + wid, p)


@functools.partial(
    pl.kernel,
    out_type=jax.ShapeDtypeStruct((B,), jnp.float32),
    mesh=_mesh,
    compiler_params=pltpu.CompilerParams(needs_layout_passes=False, use_tc_tiling_on_sc=False),
    scratch_types=[
        pltpu.VMEM((IDX_ROWS, IDX_COLS), jnp.int32),
        [pltpu.VMEM((ROWS_PER_CHUNK, D), jnp.float32) for _ in range(NBUF)],
        pltpu.VMEM((B_PER_W * F + 16,), jnp.float32),
        pltpu.VMEM((256,), jnp.float32),
        pltpu.VMEM((B_PER_W,), jnp.float32),
        pltpu.VMEM((16,), jnp.float32),
        [pltpu.SemaphoreType.DMA for _ in range(NBUF)],
        pltpu.SemaphoreType.DMA,
    ],
)
def _fm_sc(idx_hbm, fm_hbm, lr_hbm, bias_hbm, out_hbm,
           idx_v, rows_bufs, lr_v, tm_v, out_v, bias_v, sems, sem_lr):
    wid = lax.axis_index("s") * NC + lax.axis_index("c")
    pltpu.sync_copy(idx_hbm.at[wid], idx_v)
    pltpu.sync_copy(bias_hbm, bias_v.at[pl.ds(0, 1)])
    for r in range(IDX_ROWS):
        pltpu.async_copy(lr_hbm.at[idx_v.at[r]],
                         lr_v.at[pl.ds(r * IDX_COLS, IDX_COLS)], sem_lr)
    bias_s = bias_v[pl.ds(0, 16)][0]
    lane = lax.iota(jnp.int32, 16)
    mask10 = lane < 10
    zero16 = jnp.zeros((16,), jnp.float32)

    def fire(c, buf, sem):
        for j in range(GROWS):
            pltpu.async_copy(fm_hbm.at[idx_v.at[c * GROWS + j]],
                             buf.at[pl.ds(j * IDX_COLS, IDX_COLS)], sem)

    def drain(buf, sem):
        pltpu.make_async_copy(fm_hbm.at[pl.ds(0, ROWS_PER_CHUNK)], buf, sem).wait()

    def compute(c, buf):
        for g in range(CB // 16):
            for bb in range(16):
                b = g * 16 + bb
                s = zero16
                q = zero16
                for f in range(F):
                    e = buf[b * F + f]
                    s = s + e
                    q = q + e * e
                t = 0.5 * (s * s - q)
                l1 = lr_v[pl.ds(c * (CB * F) + b * F, 16)]
                l2 = jnp.where(mask10,
                               lr_v[pl.ds(c * (CB * F) + b * F + 16, 16)], 0.0)
                tm_v[pl.ds(bb * 16, 16)] = t + l1 + l2
            acc = jnp.full((16,), bias_s, jnp.float32)
            for dcol in range(16):
                acc = acc + plsc.load_gather(tm_v, [lane * 16 + dcol])
            out_v[pl.ds(c * CB + g * 16, 16)] = acc

    for p in range(NBUF):
        fire(p, rows_bufs[p], sems[p])
    pltpu.make_async_copy(lr_hbm.at[pl.ds(0, B_PER_W * F)],
                          lr_v.at[pl.ds(0, B_PER_W * F)], sem_lr).wait()

    def body(i, carry):
        c0 = i * NBUF
        for p in range(NBUF):
            c = c0 + p
            drain(rows_bufs[p], sems[p])
            compute(c, rows_bufs[p])

            @pl.when(c + NBUF < NCHUNK)
            def _():
                fire(c + NBUF, rows_bufs[p], sems[p])
        return carry

    lax.fori_loop(0, NCHUNK // NBUF, body, 0)
    pltpu.sync_copy(out_v, out_hbm.at[pl.ds(wid * B_PER_W, B_PER_W)])


def kernel(cate_indices, fm_table, lr_table, lr_bias):
    idx = cate_indices.astype(jnp.int32).reshape(NW, IDX_ROWS, IDX_COLS)
    fm_rows = _tr_sc(fm_table.T).reshape(V, D)
    lr_flat = lr_table.reshape(V)
    out = _fm_sc(idx, fm_rows, lr_flat, lr_bias)
    return out.reshape(B, 1)


# speedup vs baseline: 3.2073x; 1.0838x over previous
import functools

import jax
import jax.numpy as jnp
from jax import lax
from jax.experimental import pallas as pl
from jax.experimental.pallas import tpu as pltpu
from jax.experimental.pallas import tpu_sc as plsc

B = 16384
F = 26
V = 1000000
D = 16

NC = 2
NS = 16
NW = NC * NS
B_PER_W = B // NW
IDX_COLS = 104
IDX_ROWS = (B_PER_W * F) // IDX_COLS
CB = 32
ROWS_PER_CHUNK = CB * F
GROWS = ROWS_PER_CHUNK // IDX_COLS
NCHUNK = B_PER_W // CB
NBUF = 4

_mesh = plsc.VectorSubcoreMesh(core_axis_name="c", subcore_axis_name="s")

TTILES = (V + 127) // 128
TK = 4
TG = (TTILES + TK - 1) // TK
TGFULL = TG - 1
TSTEPS = (TG + NW - 1) // NW


@functools.partial(
    pl.kernel,
    out_type=jax.ShapeDtypeStruct((V * D,), jnp.float32),
    mesh=_mesh,
    compiler_params=pltpu.CompilerParams(needs_layout_passes=False, use_tc_tiling_on_sc=True),
    scratch_types=[
        [pltpu.VMEM((8, TK * 128), jnp.float32) for _ in range(4)],
        [pltpu.VMEM((TK * 2048,), jnp.float32) for _ in range(2)],
        [pltpu.SemaphoreType.DMA for _ in range(2)],
        [pltpu.SemaphoreType.DMA for _ in range(2)],
    ],
)
def _tr_sc(fmt_hbm, out_hbm, xbufs, ybufs, insems, outsems):
    wid = lax.axis_index("s") * NC + lax.axis_index("c")
    lane16 = lax.iota(jnp.int32, 16) * 16
    idxv = [lane16 + d for d in range(D)]

    def fire_in(g, p):
        @pl.when(g < TGFULL)
        def _():
            pltpu.async_copy(fmt_hbm.at[pl.ds(0, 8), pl.ds(g * (TK * 128), TK * 128)],
                             xbufs[2 * p], insems[p])
            pltpu.async_copy(fmt_hbm.at[pl.ds(8, 8), pl.ds(g * (TK * 128), TK * 128)],
                             xbufs[2 * p + 1], insems[p])

        @pl.when(g == TGFULL)
        def _():
            pltpu.async_copy(fmt_hbm.at[pl.ds(0, 8), pl.ds(g * (TK * 128), 128)],
                             xbufs[2 * p].at[:, pl.ds(0, 128)], insems[p])
            pltpu.async_copy(fmt_hbm.at[pl.ds(8, 8), pl.ds(g * (TK * 128), 128)],
                             xbufs[2 * p + 1].at[:, pl.ds(0, 128)], insems[p])

    def drain_in(g, p):
        @pl.when(g < TGFULL)
        def _():
            for h in range(2):
                pltpu.make_async_copy(fmt_hbm.at[pl.ds(0, 8), pl.ds(0, TK * 128)],
                                      xbufs[2 * p + h], insems[p]).wait()

        @pl.when(g == TGFULL)
        def _():
            for h in range(2):
                pltpu.make_async_copy(
                    fmt_hbm.at[pl.ds(0, 8), pl.ds(0, 128)],
                    xbufs[2 * p + h].at[:, pl.ds(0, 128)], insems[p]).wait()

    def shuffle_tile(p, t):
        y = ybufs[p]
        for d in range(D):
            src = xbufs[2 * p + (d // 8)]
            for v in range(8):
                va = src[d % 8, pl.ds(t * 128 + v * 16, 16)]
                plsc.store_scatter(y.at[pl.ds(t * 2048 + v * 256, 256)],
                                   [idxv[d]], va)

    def shuffle(g, p):
        @pl.when(g < TGFULL)
        def _():
            for t in range(TK):
                shuffle_tile(p, t)

        @pl.when(g == TGFULL)
        def _():
            shuffle_tile(p, 0)

    def write_out(g, p):
        @pl.when(g < TGFULL)
        def _():
            pltpu.async_copy(ybufs[p], out_hbm.at[pl.ds(g * (TK * 2048), TK * 2048)],
                             outsems[p])

        @pl.when(g == TGFULL)
        def _():
            pltpu.async_copy(ybufs[p].at[pl.ds(0, 1024)],
                             out_hbm.at[pl.ds(g * (TK * 2048), 1024)], outsems[p])

    def drain_out(g, p):
        @pl.when(jnp.logical_and(g >= 0, g < TGFULL))
        def _():
            pltpu.make_async_copy(ybufs[p], out_hbm.at[pl.ds(0, TK * 2048)],
                                  outsems[p]).wait()

        @pl.when(g == TGFULL)
        def _():
            pltpu.make_async_copy(ybufs[p].at[pl.ds(0, 1024)],
                                  out_hbm.at[pl.ds(0, 1024)], outsems[p]).wait()

    fire_in(wid, 0)

    def tr_body(k, carry):
        for p in range(2):
            m = 2 * k + p
            g = m * NW + wid

            @pl.when(g < TG)
            def _():
                fire_in(g + NW, 1 - p)
                drain_in(g, p)
                drain_out(g - 2 * NW, p)
                shuffle(g, p)
                write_out(g, p)
        return carry

    lax.fori_loop(0, (TSTEPS + 1) // 2, tr_body, 0)
    n_steps = (TG - 1 - wid) // NW + 1
    for p in range(2):
        m_p = n_steps - 1 - jnp.mod(n_steps - 1 - p, 2)

        @pl.when(m_p >= 0)
        def _():
            drain_out(m_p * NW + wid, p)


@functools.partial(
    pl.kernel,
    out_type=jax.ShapeDtypeStruct((B,), jnp.float32),
    mesh=_mesh,
    compiler_params=pltpu.CompilerParams(needs_layout_passes=False, use_tc_tiling_on_sc=False),
    scratch_types=[
        pltpu.VMEM((IDX_ROWS, IDX_COLS), jnp.int32),
        [pltpu.VMEM((ROWS_PER_CHUNK, D), jnp.float32) for _ in range(NBUF)],
        pltpu.VMEM((B_PER_W * F + 16,), jnp.float32),
        pltpu.VMEM((256,), jnp.float32),
        pltpu.VMEM((B_PER_W,), jnp.float32),
        pltpu.VMEM((16,), jnp.float32),
        [pltpu.SemaphoreType.DMA for _ in range(NBUF)],
        pltpu.SemaphoreType.DMA,
    ],
)
def _fm_sc(idx_hbm, fm_hbm, lr_hbm, bias_hbm, out_hbm,
           idx_v, rows_bufs, lr_v, tm_v, out_v, bias_v, sems, sem_lr):
    wid = lax.axis_index("s") * NC + lax.axis_index("c")
    pltpu.sync_copy(idx_hbm.at[wid], idx_v)
    pltpu.sync_copy(bias_hbm, bias_v.at[pl.ds(0, 1)])
    for r in range(IDX_ROWS):
        pltpu.async_copy(lr_hbm.at[idx_v.at[r]],
                         lr_v.at[pl.ds(r * IDX_COLS, IDX_COLS)], sem_lr)
    bias_s = bias_v[pl.ds(0, 16)][0]
    lane = lax.iota(jnp.int32, 16)
    mask10 = lane < 10
    zero16 = jnp.zeros((16,), jnp.float32)

    def fire(c, buf, sem):
        for j in range(GROWS):
            pltpu.async_copy(fm_hbm.at[idx_v.at[c * GROWS + j]],
                             buf.at[pl.ds(j * IDX_COLS, IDX_COLS)], sem)

    def drain(buf, sem):
        pltpu.make_async_copy(fm_hbm.at[pl.ds(0, ROWS_PER_CHUNK)], buf, sem).wait()

    def compute(c, buf):
        for g in range(CB // 16):
            for bb in range(16):
                b = g * 16 + bb
                s = zero16
                q = zero16
                for f in range(F):
                    e = buf[b * F + f]
                    s = s + e
                    q = q + e * e
                t = 0.5 * (s * s - q)
                l1 = lr_v[pl.ds(c * (CB * F) + b * F, 16)]
                l2 = jnp.where(mask10,
                               lr_v[pl.ds(c * (CB * F) + b * F + 16, 16)], 0.0)
                tm_v[pl.ds(bb * 16, 16)] = t + l1 + l2
            acc = jnp.full((16,), bias_s, jnp.float32)
            for dcol in range(16):
                acc = acc + plsc.load_gather(tm_v, [lane * 16 + dcol])
            out_v[pl.ds(c * CB + g * 16, 16)] = acc

    for p in range(NBUF):
        fire(p, rows_bufs[p], sems[p])
    pltpu.make_async_copy(lr_hbm.at[pl.ds(0, B_PER_W * F)],
                          lr_v.at[pl.ds(0, B_PER_W * F)], sem_lr).wait()

    def body(i, carry):
        c0 = i * NBUF
        for p in range(NBUF):
            c = c0 + p
            drain(rows_bufs[p], sems[p])
            compute(c, rows_bufs[p])

            @pl.when(c + NBUF < NCHUNK)
            def _():
                fire(c + NBUF, rows_bufs[p], sems[p])
        return carry

    lax.fori_loop(0, NCHUNK // NBUF, body, 0)
    pltpu.sync_copy(out_v, out_hbm.at[pl.ds(wid * B_PER_W, B_PER_W)])


def kernel(cate_indices, fm_table, lr_table, lr_bias):
    idx = cate_indices.astype(jnp.int32).reshape(NW, IDX_ROWS, IDX_COLS)
    fm_rows = _tr_sc(fm_table.T).reshape(V, D)
    lr_flat = lr_table.reshape(V)
    out = _fm_sc(idx, fm_rows, lr_flat, lr_bias)
    return out.reshape(B, 1)
